# baseline (speedup 1.0000x reference)
import jax
import jax.numpy as jnp
from jax.experimental import pallas as pl
from jax.experimental.pallas import tpu as pltpu

NCLS = 4
CIN = 768
HH = 32
WW = 32
HID = 256
P = HH * WW
KOUT = NCLS * HID
NTAPS = 9
KW = NTAPS * CIN
JB = 128
NJ = KOUT // JB


def _body(xt_ref, w_ref, b1_ref, w2t_ref, b2_ref, out_ref, xc_ref, acc_ref):
    j = pl.program_id(0)

    @pl.when(j == 0)
    def _init():
        x = xt_ref[...]
        ss = jnp.sum(x * x, axis=1, keepdims=True)
        xn = x / jnp.maximum(jnp.sqrt(ss), 1e-12)
        pix = jax.lax.broadcasted_iota(jnp.int32, (P, 1), 0) % WW
        xs = (jnp.where(pix != WW - 1, xn, 0.0), xn,
              jnp.where(pix != 0, xn, 0.0))
        for k in range(NTAPS):
            dy = k // 3 - 1
            dx = k % 3 - 1
            s = dy * WW + dx
            lo = max(0, -s)
            hi = min(P, P - s)
            c0 = k * CIN
            if lo > 0:
                xc_ref[:lo, c0:c0 + CIN] = jnp.zeros((lo, CIN), jnp.float32)
            xc_ref[lo:hi, c0:c0 + CIN] = xs[dx + 1][lo + s:hi + s, :]
            if hi < P:
                xc_ref[hi:, c0:c0 + CIN] = jnp.zeros((P - hi, CIN),
                                                     jnp.float32)

    yt = jax.lax.dot_general(xc_ref[...], w_ref[...], (((1,), (1,)), ((), ())),
                             preferred_element_type=jnp.float32)
    h = jnp.maximum(yt + b1_ref[...], 0.0)
    part = jnp.dot(h, w2t_ref[...],
                   preferred_element_type=jnp.float32)

    @pl.when(j == 0)
    def _acc0():
        acc_ref[...] = part

    @pl.when(j > 0)
    def _acc():
        acc_ref[...] += part

    @pl.when(j == NJ - 1)
    def _tail():
        logits = acc_ref[...] + b2_ref[...]
        probs = jax.nn.sigmoid(logits)
        maxp = jnp.max(probs, axis=1, keepdims=True)
        anyd = jnp.max(logits, axis=1, keepdims=True) > 0.0
        bg = jnp.where(anyd, 0.0, 1.0 - maxp)
        out_ref[...] = jnp.concatenate([bg, probs], axis=1)


def kernel(query_feat, W1, b1, W2, b2):
    xt = query_feat.reshape(CIN, P).T
    wt = jnp.transpose(W1.reshape(KOUT, CIN, NTAPS),
                       (0, 2, 1)).reshape(KOUT, KW)
    b1r = b1.reshape(1, KOUT)
    w2t = (jnp.eye(NCLS, dtype=jnp.float32)[:, None, :]
           * W2.reshape(NCLS, HID, 1)).reshape(KOUT, NCLS)
    b2r = b2.reshape(1, NCLS)

    out = pl.pallas_call(
        _body,
        grid=(NJ,),
        in_specs=[
            pl.BlockSpec((P, CIN), lambda j: (0, 0)),
            pl.BlockSpec((JB, KW), lambda j: (j, 0)),
            pl.BlockSpec((1, JB), lambda j: (0, j)),
            pl.BlockSpec((JB, NCLS), lambda j: (j, 0)),
            pl.BlockSpec((1, NCLS), lambda j: (0, 0)),
        ],
        out_specs=pl.BlockSpec((P, 1 + NCLS), lambda j: (0, 0)),
        out_shape=jax.ShapeDtypeStruct((P, 1 + NCLS), jnp.float32),
        scratch_shapes=[
            pltpu.VMEM((P, KW), jnp.float32),
            pltpu.VMEM((P, NCLS), jnp.float32),
        ],
    )(xt, wt, b1r, w2t, b2r)

    return out.T.reshape(1, 1 + NCLS, HH, WW)
